# baseline (speedup 1.0000x reference)
import jax
import jax.numpy as jnp
from jax.experimental import pallas as pl
from jax.experimental.pallas import tpu as pltpu

G = 5
E = 5
J = 5
T = 9
D = 12
JD = J * D
IN = T * JD
W = G * JD
H1 = 512
H2 = 256
ZD = 64
B = 1024
BT = 512
NB = B // BT


def _moe_kernel(x_ref, W1_ref, b1_ref, W2_ref, b2_ref, Wmu_ref, bmu_ref,
                Wlv_ref, blv_ref, Wd1_ref, bd1_ref, Wd2_ref, bd2_ref,
                Wd3_ref, bd3_ref, mu_ref, lv_ref, xh_ref, idx_ref,
                xf_s, err_s, xhb_s):
    g = pl.program_id(1)
    e = pl.program_id(2)

    @pl.when(e == 0)
    def _gather():
        for gg in range(G):
            @pl.when(g == gg)
            def _():
                for t in range(T):
                    xf_s[:, t * JD:(t + 1) * JD] = \
                        x_ref[:, t * W + gg * JD: t * W + (gg + 1) * JD]

    xfb = xf_s[...]

    h1 = jax.nn.relu(jnp.dot(xfb, W1_ref[0, 0]) + b1_ref[0, 0])
    h2 = jax.nn.relu(jnp.dot(h1, W2_ref[0, 0]) + b2_ref[0, 0])
    mu = jnp.dot(h2, Wmu_ref[0, 0]) + bmu_ref[0, 0]
    lv = jnp.dot(h2, Wlv_ref[0, 0]) + blv_ref[0, 0]
    d1 = jax.nn.relu(jnp.dot(mu, Wd1_ref[0, 0]) + bd1_ref[0, 0])
    d2 = jax.nn.relu(jnp.dot(d1, Wd2_ref[0, 0]) + bd2_ref[0, 0])
    xh = jnp.dot(d2, Wd3_ref[0, 0]) + bd3_ref[0, 0]
    diff = xh - xfb
    err = jnp.mean(diff * diff, axis=-1, keepdims=True)

    @pl.when(e == 0)
    def _init():
        mu_ref[0] = mu
        lv_ref[0] = lv
        idx_ref[0] = jnp.zeros((BT, 1), dtype=jnp.int32)
        err_s[...] = err
        xhb_s[...] = xh

    @pl.when(e > 0)
    def _select():
        better = err < err_s[...]
        mu_ref[0] = jnp.where(better, mu, mu_ref[0])
        lv_ref[0] = jnp.where(better, lv, lv_ref[0])
        idx_ref[0] = jnp.where(better, e.astype(jnp.int32), idx_ref[0])
        err_s[...] = jnp.where(better, err, err_s[...])
        xhb_s[...] = jnp.where(better, xh, xhb_s[...])

    @pl.when(e == E - 1)
    def _scatter():
        xhb = xhb_s[...]
        for gg in range(G):
            @pl.when(g == gg)
            def _():
                for t in range(T):
                    xh_ref[:, t * W + gg * JD: t * W + (gg + 1) * JD] = \
                        xhb[:, t * JD:(t + 1) * JD]


def kernel(x, W1, b1, W2, b2, Wmu, bmu, Wlv, blv, Wd1, bd1, Wd2, bd2, Wd3, bd3):
    Bb = x.shape[0]
    nb = Bb // BT
    xflat = x.reshape(Bb, T * W)

    wspec = lambda *s: pl.BlockSpec((1, 1) + s, lambda b, g, e: (g, e) + (0,) * len(s))
    bias = lambda a: a.reshape(G, E, 1, a.shape[-1])
    bspec = lambda n: pl.BlockSpec((1, 1, 1, n), lambda b, g, e: (g, e, 0, 0))

    mu_sel, lv_sel, xh_flat, idx = pl.pallas_call(
        _moe_kernel,
        grid=(nb, G, E),
        in_specs=[
            pl.BlockSpec((BT, T * W), lambda b, g, e: (b, 0)),
            wspec(IN, H1), bspec(H1),
            wspec(H1, H2), bspec(H2),
            wspec(H2, ZD), bspec(ZD),
            wspec(H2, ZD), bspec(ZD),
            wspec(ZD, H2), bspec(H2),
            wspec(H2, H1), bspec(H1),
            wspec(H1, IN), bspec(IN),
        ],
        out_specs=[
            pl.BlockSpec((1, BT, ZD), lambda b, g, e: (g, b, 0)),
            pl.BlockSpec((1, BT, ZD), lambda b, g, e: (g, b, 0)),
            pl.BlockSpec((BT, T * W), lambda b, g, e: (b, 0)),
            pl.BlockSpec((1, BT, 1), lambda b, g, e: (g, b, 0)),
        ],
        out_shape=[
            jax.ShapeDtypeStruct((G, Bb, ZD), jnp.float32),
            jax.ShapeDtypeStruct((G, Bb, ZD), jnp.float32),
            jax.ShapeDtypeStruct((Bb, T * W), jnp.float32),
            jax.ShapeDtypeStruct((G, Bb, 1), jnp.int32),
        ],
        scratch_shapes=[
            pltpu.VMEM((BT, IN), jnp.float32),
            pltpu.VMEM((BT, 1), jnp.float32),
            pltpu.VMEM((BT, IN), jnp.float32),
        ],
    )(xflat, W1, bias(b1), W2, bias(b2), Wmu, bias(bmu), Wlv, bias(blv),
      Wd1, bias(bd1), Wd2, bias(bd2), Wd3, bias(bd3))

    xhat = xh_flat.reshape(Bb, T, G * J, D)
    return mu_sel, lv_sel, xhat, idx[:, :, 0]
